# baseline (speedup 1.0000x reference)
import functools

import jax
import jax.numpy as jnp
from jax.experimental import pallas as pl
from jax.experimental.pallas import tpu as pltpu


def _conv_stack_kernel(x_ref, w1_ref, b1_ref, w2_ref, b2_ref, o_ref, *,
                       h, w, cin, ch):
    f32 = jnp.float32
    bf16 = jnp.bfloat16
    B = x_ref.shape[0]
    wcin = w * cin
    wch = w * ch
    h2 = h // 2
    w2c = (w // 2) * ch
    M1 = B * h

    X = x_ref[...].reshape(M1, wcin)
    rows = jax.lax.broadcasted_iota(jnp.int32, (M1, wcin), 0)
    z1 = jnp.zeros((1, wcin), f32)
    Xd = jnp.where(rows % h == 0, 0.0, jnp.concatenate([z1, X[:-1]], axis=0))
    Xu = jnp.where(rows % h == h - 1, 0.0,
                   jnp.concatenate([X[1:], z1], axis=0))
    X3 = jnp.concatenate([Xd, X, Xu], axis=1).astype(bf16)
    acc1 = jnp.dot(X3, w1_ref[...], preferred_element_type=f32)
    acc1 = jnp.maximum(acc1 + b1_ref[...], 0.0)

    rm = jnp.max(acc1.reshape(M1 // 2, 2, wch), axis=1)
    m1 = jnp.maximum(rm, pltpu.roll(rm, wch - ch, axis=1)).astype(bf16)

    M2 = M1 // 2
    out = jnp.dot(m1, w2_ref[...], preferred_element_type=f32)
    o0 = out[:, :w2c]
    o1 = out[:, w2c:2 * w2c]
    o2 = out[:, 2 * w2c:]
    rows2 = jax.lax.broadcasted_iota(jnp.int32, (M2, w2c), 0)
    z2 = jnp.zeros((1, w2c), f32)
    acc2 = (o1
            + jnp.where(rows2 % h2 == 0, 0.0,
                        jnp.concatenate([z2, o0[:-1]], axis=0))
            + jnp.where(rows2 % h2 == h2 - 1, 0.0,
                        jnp.concatenate([o2[1:], z2], axis=0)))
    acc2 = jnp.maximum(acc2 + b2_ref[...], 0.0)

    rm2 = jnp.max(acc2.reshape(M2 // 2, 2, w2c), axis=1)
    m2 = jnp.maximum(rm2, pltpu.roll(rm2, w2c - ch, axis=1))
    o_ref[...] = m2.reshape(B, h // 4, w2c).astype(o_ref.dtype)


def _mlp_kernel(x_ref, w1_ref, b1_ref, w2_ref, b2_ref, o_ref):
    f32 = jnp.float32
    hid = jnp.dot(x_ref[...], w1_ref[...], preferred_element_type=f32)
    hid = jnp.maximum(hid + b1_ref[...], 0.0).astype(jnp.bfloat16)
    out = jnp.dot(hid, w2_ref[...], preferred_element_type=f32) + b2_ref[...]
    o_ref[...] = out


def _forward(x, bw1, b1row, bw2, b2row, w1p, b1p, w2p, b2p, *, num_classes):
    n, cin, h, w = x.shape
    wch = b1row.shape[1]
    ch = wch // w
    w2c = b2row.shape[1]
    wp = w // 2
    h4, w4 = h // 4, w // 4
    hp = w1p.shape[1]
    cp = w2p.shape[1]
    bf16 = jnp.bfloat16

    xt = jnp.transpose(x, (0, 2, 1, 3)).reshape(n, h, cin * w)
    w1cat = jnp.transpose(bw1, (1, 0, 2, 3)).reshape(3 * cin * w, wch)
    w1cat = w1cat.astype(bf16)
    w2r = bw2.reshape(3, wp, ch, w2c)
    w2e = jnp.stack([w2r, jnp.zeros_like(w2r)], axis=2)
    w2big = w2e.reshape(3, wch, w2c).transpose(1, 0, 2).reshape(wch, 3 * w2c)
    w2big = w2big.astype(bf16)
    w1r = w1p.reshape(h4, w4, ch, hp)
    w1e = jnp.stack([w1r, jnp.zeros_like(w1r)], axis=2)
    w1exp = w1e.reshape(h4 * w2c, hp).astype(bf16)

    B = next(b for b in (16, 8, 4, 2, 1) if n % b == 0)
    feats = pl.pallas_call(
        functools.partial(_conv_stack_kernel, h=h, w=w, cin=cin, ch=ch),
        out_shape=jax.ShapeDtypeStruct((n, h4, w2c), bf16),
        grid=(n // B,),
        in_specs=[
            pl.BlockSpec((B, h, cin * w), lambda i: (i, 0, 0)),
            pl.BlockSpec((3 * cin * w, wch), lambda i: (0, 0)),
            pl.BlockSpec((1, wch), lambda i: (0, 0)),
            pl.BlockSpec((wch, 3 * w2c), lambda i: (0, 0)),
            pl.BlockSpec((1, w2c), lambda i: (0, 0)),
        ],
        out_specs=pl.BlockSpec((B, h4, w2c), lambda i: (i, 0, 0)),
        compiler_params=pltpu.CompilerParams(
            dimension_semantics=("parallel",)),
    )(xt, w1cat, b1row, w2big, b2row)

    flat = feats.reshape(n, h4 * w2c)
    mt = 128 if n % 128 == 0 else n
    logits = pl.pallas_call(
        _mlp_kernel,
        out_shape=jax.ShapeDtypeStruct((n, cp), jnp.float32),
        grid=(n // mt,),
        in_specs=[
            pl.BlockSpec((mt, h4 * w2c), lambda i: (i, 0)),
            pl.BlockSpec((h4 * w2c, hp), lambda i: (0, 0)),
            pl.BlockSpec((1, hp), lambda i: (0, 0)),
            pl.BlockSpec((hp, cp), lambda i: (0, 0)),
            pl.BlockSpec((1, cp), lambda i: (0, 0)),
        ],
        out_specs=pl.BlockSpec((mt, cp), lambda i: (i, 0)),
        compiler_params=pltpu.CompilerParams(
            dimension_semantics=("parallel",)),
    )(flat, w1exp, b1p, w2p.astype(bf16), b2p)
    return {"out": logits[:, :num_classes]}


def kernel(x, bw1, b1row, bw2, b2row, w1p, b1p, w2p, b2p):
    return _forward(x, bw1, b1row, bw2, b2row, w1p, b1p, w2p, b2p,
                    num_classes=100)


# speedup vs baseline: 3.2187x; 3.2187x over previous
import functools

import jax
import jax.numpy as jnp
from jax.experimental import pallas as pl
from jax.experimental.pallas import tpu as pltpu


def _conv_stack_kernel(x_ref, w1_ref, b1_ref, w2_ref, b2_ref, o_ref, *,
                       h, w, cin, ch):
    f32 = jnp.float32
    bf16 = jnp.bfloat16
    B = x_ref.shape[0]
    wcin = w * cin
    wch = w * ch
    h2, h4 = h // 2, h // 4
    w2c = (w // 2) * ch
    wqc = (w // 4) * ch
    M1, M2 = B * h, B * h2

    X4 = x_ref[...].reshape(B, 4, h4, wcin)

    def sd(Y):
        z = jnp.zeros((B, 1, Y.shape[-1]), f32)
        return jnp.concatenate([z, Y[:, :-1, :]], axis=1)

    def su(Y):
        z = jnp.zeros((B, 1, Y.shape[-1]), f32)
        return jnp.concatenate([Y[:, 1:, :], z], axis=1)

    Xd = jnp.stack([sd(X4[:, 3]), X4[:, 2], X4[:, 0], X4[:, 1]],
                   axis=1).reshape(M1, wcin)
    Xu = jnp.stack([X4[:, 2], X4[:, 3], X4[:, 1], su(X4[:, 0])],
                   axis=1).reshape(M1, wcin)
    X = X4.reshape(M1, wcin)
    X3 = jnp.concatenate([Xd, X, Xu], axis=1).astype(bf16)

    acc1 = jnp.dot(X3, w1_ref[...], preferred_element_type=f32)
    acc1 = jnp.maximum(acc1 + b1_ref[...], 0.0).reshape(B, 2, h2, wch)
    rm = jnp.maximum(acc1[:, 0], acc1[:, 1])
    m1 = jnp.maximum(rm[..., :w2c], rm[..., w2c:]).astype(bf16)

    out = jnp.dot(m1.reshape(M2, w2c), w2_ref[...],
                  preferred_element_type=f32)
    o0 = out[:, :w2c].reshape(B, 2, h4, w2c)
    o1 = out[:, w2c:2 * w2c].reshape(B, 2, h4, w2c)
    o2 = out[:, 2 * w2c:].reshape(B, 2, h4, w2c)
    dpart = jnp.stack([sd(o0[:, 1]), o0[:, 0]], axis=1)
    upart = jnp.stack([o2[:, 1], su(o2[:, 0])], axis=1)
    acc2 = jnp.maximum(o1 + dpart + upart + b2_ref[...], 0.0)
    rm2 = jnp.maximum(acc2[:, 0], acc2[:, 1])
    m2 = jnp.maximum(rm2[..., :wqc], rm2[..., wqc:])
    o_ref[...] = m2.astype(o_ref.dtype)


def _mlp_kernel(x_ref, w1_ref, b1_ref, w2_ref, b2_ref, o_ref):
    f32 = jnp.float32
    hid = jnp.dot(x_ref[...], w1_ref[...], preferred_element_type=f32)
    hid = jnp.maximum(hid + b1_ref[...], 0.0).astype(jnp.bfloat16)
    out = jnp.dot(hid, w2_ref[...], preferred_element_type=f32) + b2_ref[...]
    o_ref[...] = out


def _colperm(a, npix, ch):
    lead = a.shape[:-1]
    a = a.reshape(*lead, npix // 2, 2, ch)
    a = jnp.swapaxes(a, -3, -2)
    return a.reshape(*lead, npix * ch)


def _forward(x, bw1, b1row, bw2, b2row, w1p, b1p, w2p, b2p, *, num_classes):
    n, cin, h, w = x.shape
    wch = b1row.shape[1]
    ch = wch // w
    w2c = b2row.shape[1]
    wp = w // 2
    h4 = h // 4
    wqc = (w // 4) * ch
    hp = w1p.shape[1]
    cp = w2p.shape[1]
    bf16 = jnp.bfloat16

    xt = jnp.transpose(x, (0, 2, 1, 3)).reshape(n, h, cin * w)
    xt = xt.reshape(n, h4, 2, 2, cin * w).transpose(0, 3, 2, 1, 4)
    xt = xt.reshape(n, h, cin * w)
    w1cat = jnp.transpose(bw1, (1, 0, 2, 3)).reshape(3 * cin * w, wch)
    w1cat = _colperm(w1cat, w, ch).astype(bf16)
    b1c = _colperm(b1row, w, ch)
    w2c3 = _colperm(bw2, wp, ch)
    w2big = w2c3.transpose(1, 0, 2).reshape(w2c, 3 * w2c).astype(bf16)
    b2c = _colperm(b2row, wp, ch)

    B = next(b for b in (16, 8, 4, 2, 1) if n % b == 0)
    feats = pl.pallas_call(
        functools.partial(_conv_stack_kernel, h=h, w=w, cin=cin, ch=ch),
        out_shape=jax.ShapeDtypeStruct((n, h4, wqc), bf16),
        grid=(n // B,),
        in_specs=[
            pl.BlockSpec((B, h, cin * w), lambda i: (i, 0, 0)),
            pl.BlockSpec((3 * cin * w, wch), lambda i: (0, 0)),
            pl.BlockSpec((1, wch), lambda i: (0, 0)),
            pl.BlockSpec((w2c, 3 * w2c), lambda i: (0, 0)),
            pl.BlockSpec((1, w2c), lambda i: (0, 0)),
        ],
        out_specs=pl.BlockSpec((B, h4, wqc), lambda i: (i, 0, 0)),
        compiler_params=pltpu.CompilerParams(
            dimension_semantics=("parallel",)),
    )(xt, w1cat, b1c, w2big, b2c)

    flat = feats.reshape(n, h4 * wqc)
    mt = 128 if n % 128 == 0 else n
    logits = pl.pallas_call(
        _mlp_kernel,
        out_shape=jax.ShapeDtypeStruct((n, cp), jnp.float32),
        grid=(n // mt,),
        in_specs=[
            pl.BlockSpec((mt, h4 * wqc), lambda i: (i, 0)),
            pl.BlockSpec((h4 * wqc, hp), lambda i: (0, 0)),
            pl.BlockSpec((1, hp), lambda i: (0, 0)),
            pl.BlockSpec((hp, cp), lambda i: (0, 0)),
            pl.BlockSpec((1, cp), lambda i: (0, 0)),
        ],
        out_specs=pl.BlockSpec((mt, cp), lambda i: (i, 0)),
        compiler_params=pltpu.CompilerParams(
            dimension_semantics=("parallel",)),
    )(flat, w1p.astype(bf16), b1p, w2p.astype(bf16), b2p)
    return {"out": logits[:, :num_classes]}


def kernel(x, bw1, b1row, bw2, b2row, w1p, b1p, w2p, b2p):
    return _forward(x, bw1, b1row, bw2, b2row, w1p, b1p, w2p, b2p,
                    num_classes=100)


# speedup vs baseline: 3.2282x; 1.0030x over previous
import functools

import jax
import jax.numpy as jnp
from jax.experimental import pallas as pl
from jax.experimental.pallas import tpu as pltpu


def _conv_stack_kernel(x_ref, w1_ref, b1_ref, w2_ref, b2_ref, o_ref, *,
                       h, w, cin, ch):
    f32 = jnp.float32
    bf16 = jnp.bfloat16
    B = x_ref.shape[0]
    wcin = w * cin
    wch = w * ch
    h2, h4 = h // 2, h // 4
    w2c = (w // 2) * ch
    wqc = (w // 4) * ch
    M1, M2 = B * h, B * h2

    X4 = x_ref[...].reshape(B, 4, h4, wcin)

    def sd(Y):
        z = jnp.zeros((B, 1, Y.shape[-1]), f32)
        return jnp.concatenate([z, Y[:, :-1, :]], axis=1)

    def su(Y):
        z = jnp.zeros((B, 1, Y.shape[-1]), f32)
        return jnp.concatenate([Y[:, 1:, :], z], axis=1)

    Xd = jnp.stack([sd(X4[:, 3]), X4[:, 2], X4[:, 0], X4[:, 1]],
                   axis=1).reshape(M1, wcin)
    Xu = jnp.stack([X4[:, 2], X4[:, 3], X4[:, 1], su(X4[:, 0])],
                   axis=1).reshape(M1, wcin)
    X = X4.reshape(M1, wcin)
    X3 = jnp.concatenate([Xd, X, Xu], axis=1).astype(bf16)

    acc1 = jnp.dot(X3, w1_ref[...], preferred_element_type=f32)
    acc1 = jnp.maximum(acc1 + b1_ref[...], 0.0).reshape(B, 2, h2, wch)
    rm = jnp.maximum(acc1[:, 0], acc1[:, 1])
    m1 = jnp.maximum(rm[..., :w2c], rm[..., w2c:]).astype(bf16)

    out = jnp.dot(m1.reshape(M2, w2c), w2_ref[...],
                  preferred_element_type=f32)
    o0 = out[:, :w2c].reshape(B, 2, h4, w2c)
    o1 = out[:, w2c:2 * w2c].reshape(B, 2, h4, w2c)
    o2 = out[:, 2 * w2c:].reshape(B, 2, h4, w2c)
    dpart = jnp.stack([sd(o0[:, 1]), o0[:, 0]], axis=1)
    upart = jnp.stack([o2[:, 1], su(o2[:, 0])], axis=1)
    acc2 = jnp.maximum(o1 + dpart + upart + b2_ref[...], 0.0)
    rm2 = jnp.maximum(acc2[:, 0], acc2[:, 1])
    m2 = jnp.maximum(rm2[..., :wqc], rm2[..., wqc:])
    o_ref[...] = m2.astype(o_ref.dtype)


def _mlp_kernel(x_ref, w1_ref, b1_ref, w2_ref, b2_ref, o_ref):
    f32 = jnp.float32
    hid = jnp.dot(x_ref[...], w1_ref[...], preferred_element_type=f32)
    hid = jnp.maximum(hid + b1_ref[...], 0.0).astype(jnp.bfloat16)
    out = jnp.dot(hid, w2_ref[...], preferred_element_type=f32) + b2_ref[...]
    o_ref[...] = out


def _colperm(a, npix, ch):
    lead = a.shape[:-1]
    a = a.reshape(*lead, npix // 2, 2, ch)
    a = jnp.swapaxes(a, -3, -2)
    return a.reshape(*lead, npix * ch)


def _forward(x, bw1, b1row, bw2, b2row, w1p, b1p, w2p, b2p, *, num_classes):
    n, cin, h, w = x.shape
    wch = b1row.shape[1]
    ch = wch // w
    w2c = b2row.shape[1]
    wp = w // 2
    h4 = h // 4
    wqc = (w // 4) * ch
    hp = w1p.shape[1]
    cp = w2p.shape[1]
    bf16 = jnp.bfloat16

    xt = x.reshape(n, cin, h4, 2, 2, w).transpose(0, 4, 3, 2, 1, 5)
    xt = xt.reshape(n, h, cin * w)
    w1cat = jnp.transpose(bw1.astype(bf16), (1, 0, 2, 3))
    w1cat = _colperm(w1cat.reshape(3 * cin * w, wch), w, ch)
    b1c = _colperm(b1row, w, ch)
    w2c3 = _colperm(bw2.astype(bf16), wp, ch)
    w2big = w2c3.transpose(1, 0, 2).reshape(w2c, 3 * w2c)
    b2c = _colperm(b2row, wp, ch)

    B = next(b for b in (32, 16, 8, 4, 2, 1) if n % b == 0)
    feats = pl.pallas_call(
        functools.partial(_conv_stack_kernel, h=h, w=w, cin=cin, ch=ch),
        out_shape=jax.ShapeDtypeStruct((n, h4, wqc), bf16),
        grid=(n // B,),
        in_specs=[
            pl.BlockSpec((B, h, cin * w), lambda i: (i, 0, 0)),
            pl.BlockSpec((3 * cin * w, wch), lambda i: (0, 0)),
            pl.BlockSpec((1, wch), lambda i: (0, 0)),
            pl.BlockSpec((w2c, 3 * w2c), lambda i: (0, 0)),
            pl.BlockSpec((1, w2c), lambda i: (0, 0)),
        ],
        out_specs=pl.BlockSpec((B, h4, wqc), lambda i: (i, 0, 0)),
        compiler_params=pltpu.CompilerParams(
            dimension_semantics=("parallel",)),
    )(xt, w1cat, b1c, w2big, b2c)

    flat = feats.reshape(n, h4 * wqc)
    mt = 128 if n % 128 == 0 else n
    logits = pl.pallas_call(
        _mlp_kernel,
        out_shape=jax.ShapeDtypeStruct((n, cp), jnp.float32),
        grid=(n // mt,),
        in_specs=[
            pl.BlockSpec((mt, h4 * wqc), lambda i: (i, 0)),
            pl.BlockSpec((h4 * wqc, hp), lambda i: (0, 0)),
            pl.BlockSpec((1, hp), lambda i: (0, 0)),
            pl.BlockSpec((hp, cp), lambda i: (0, 0)),
            pl.BlockSpec((1, cp), lambda i: (0, 0)),
        ],
        out_specs=pl.BlockSpec((mt, cp), lambda i: (i, 0)),
        compiler_params=pltpu.CompilerParams(
            dimension_semantics=("parallel",)),
    )(flat, w1p.astype(bf16), b1p, w2p.astype(bf16), b2p)
    return {"out": logits[:, :num_classes]}


def kernel(x, bw1, b1row, bw2, b2row, w1p, b1p, w2p, b2p):
    return _forward(x, bw1, b1row, bw2, b2row, w1p, b1p, w2p, b2p,
                    num_classes=100)


# speedup vs baseline: 3.3835x; 1.0481x over previous
import functools

import jax
import jax.numpy as jnp
from jax.experimental import pallas as pl
from jax.experimental.pallas import tpu as pltpu


def _conv_stack_kernel(x_ref, w1_ref, b1_ref, w2_ref, b2_ref, o_ref, *,
                       h, w, cin, ch):
    f32 = jnp.float32
    bf16 = jnp.bfloat16
    B = x_ref.shape[0]
    wcin = w * cin
    wch = w * ch
    h2, h4 = h // 2, h // 4
    w2c = (w // 2) * ch
    wqc = (w // 4) * ch
    M1, M2 = B * h, B * h2

    X4 = x_ref[...].reshape(B, 4, h4, wcin)

    def sd(Y):
        z = jnp.zeros((B, 1, Y.shape[-1]), Y.dtype)
        return jnp.concatenate([z, Y[:, :-1, :]], axis=1)

    def su(Y):
        z = jnp.zeros((B, 1, Y.shape[-1]), Y.dtype)
        return jnp.concatenate([Y[:, 1:, :], z], axis=1)

    Xd = jnp.stack([sd(X4[:, 3]), X4[:, 2], X4[:, 0], X4[:, 1]],
                   axis=1).reshape(M1, wcin)
    Xu = jnp.stack([X4[:, 2], X4[:, 3], X4[:, 1], su(X4[:, 0])],
                   axis=1).reshape(M1, wcin)
    X = X4.reshape(M1, wcin)
    X3 = jnp.concatenate([Xd, X, Xu], axis=1).astype(bf16)

    acc1 = jnp.dot(X3, w1_ref[...], preferred_element_type=f32)
    acc1 = jnp.maximum(acc1 + b1_ref[...], 0.0).reshape(B, 2, h2, wch)
    rm = jnp.maximum(acc1[:, 0], acc1[:, 1])
    m1 = jnp.maximum(rm[..., :w2c], rm[..., w2c:]).astype(bf16)

    out = jnp.dot(m1.reshape(M2, w2c), w2_ref[...],
                  preferred_element_type=f32)
    o0 = out[:, :w2c].reshape(B, 2, h4, w2c)
    o1 = out[:, w2c:2 * w2c].reshape(B, 2, h4, w2c)
    o2 = out[:, 2 * w2c:].reshape(B, 2, h4, w2c)
    dpart = jnp.stack([sd(o0[:, 1]), o0[:, 0]], axis=1)
    upart = jnp.stack([o2[:, 1], su(o2[:, 0])], axis=1)
    acc2 = jnp.maximum(o1 + dpart + upart + b2_ref[...], 0.0)
    rm2 = jnp.maximum(acc2[:, 0], acc2[:, 1])
    m2 = jnp.maximum(rm2[..., :wqc], rm2[..., wqc:])
    o_ref[...] = m2.astype(o_ref.dtype)


def _mlp_kernel(x_ref, w1_ref, b1_ref, w2_ref, b2_ref, o_ref):
    f32 = jnp.float32
    w1 = w1_ref[...].astype(jnp.bfloat16)
    hid = jnp.dot(x_ref[...], w1, preferred_element_type=f32)
    hid = jnp.maximum(hid + b1_ref[...], 0.0).astype(jnp.bfloat16)
    out = jnp.dot(hid, w2_ref[...], preferred_element_type=f32) + b2_ref[...]
    o_ref[...] = out


def _colperm(a, npix, ch):
    lead = a.shape[:-1]
    a = a.reshape(*lead, npix // 2, 2, ch)
    a = jnp.swapaxes(a, -3, -2)
    return a.reshape(*lead, npix * ch)


def _forward(x, bw1, b1row, bw2, b2row, w1p, b1p, w2p, b2p, *, num_classes):
    n, cin, h, w = x.shape
    wch = b1row.shape[1]
    ch = wch // w
    w2c = b2row.shape[1]
    wp = w // 2
    h4 = h // 4
    wqc = (w // 4) * ch
    hp = w1p.shape[1]
    cp = w2p.shape[1]
    bf16 = jnp.bfloat16

    xt = x.reshape(n, cin, h4, 2, 2, w).transpose(0, 4, 3, 2, 1, 5)
    xt = xt.reshape(n, h, cin * w).astype(bf16)
    w1cat = jnp.transpose(bw1.astype(bf16), (1, 0, 2, 3))
    w1cat = _colperm(w1cat.reshape(3 * cin * w, wch), w, ch)
    b1c = _colperm(b1row, w, ch)
    w2c3 = _colperm(bw2.astype(bf16), wp, ch)
    w2big = w2c3.transpose(1, 0, 2).reshape(w2c, 3 * w2c)
    b2c = _colperm(b2row, wp, ch)

    B = next(b for b in (32, 16, 8, 4, 2, 1) if n % b == 0)
    feats = pl.pallas_call(
        functools.partial(_conv_stack_kernel, h=h, w=w, cin=cin, ch=ch),
        out_shape=jax.ShapeDtypeStruct((n, h4, wqc), bf16),
        grid=(n // B,),
        in_specs=[
            pl.BlockSpec((B, h, cin * w), lambda i: (i, 0, 0)),
            pl.BlockSpec((3 * cin * w, wch), lambda i: (0, 0)),
            pl.BlockSpec((1, wch), lambda i: (0, 0)),
            pl.BlockSpec((w2c, 3 * w2c), lambda i: (0, 0)),
            pl.BlockSpec((1, w2c), lambda i: (0, 0)),
        ],
        out_specs=pl.BlockSpec((B, h4, wqc), lambda i: (i, 0, 0)),
        compiler_params=pltpu.CompilerParams(
            dimension_semantics=("parallel",)),
    )(xt, w1cat, b1c, w2big, b2c)

    flat = feats.reshape(n, h4 * wqc)
    mt = n
    logits = pl.pallas_call(
        _mlp_kernel,
        out_shape=jax.ShapeDtypeStruct((n, cp), jnp.float32),
        grid=(n // mt,),
        in_specs=[
            pl.BlockSpec((mt, h4 * wqc), lambda i: (i, 0)),
            pl.BlockSpec((h4 * wqc, hp), lambda i: (0, 0)),
            pl.BlockSpec((1, hp), lambda i: (0, 0)),
            pl.BlockSpec((hp, cp), lambda i: (0, 0)),
            pl.BlockSpec((1, cp), lambda i: (0, 0)),
        ],
        out_specs=pl.BlockSpec((mt, cp), lambda i: (i, 0)),
        compiler_params=pltpu.CompilerParams(
            dimension_semantics=("parallel",)),
    )(flat, w1p, b1p, w2p.astype(bf16), b2p)
    return {"out": logits[:, :num_classes]}


def kernel(x, bw1, b1row, bw2, b2row, w1p, b1p, w2p, b2p):
    return _forward(x, bw1, b1row, bw2, b2row, w1p, b1p, w2p, b2p,
                    num_classes=100)


# speedup vs baseline: 3.3848x; 1.0004x over previous
import functools

import jax
import jax.numpy as jnp
from jax.experimental import pallas as pl
from jax.experimental.pallas import tpu as pltpu


def _conv_stack_kernel(x_ref, w1_ref, b1_ref, w2_ref, b2_ref, o_ref, *,
                       h, w, cin, ch):
    f32 = jnp.float32
    bf16 = jnp.bfloat16
    B = x_ref.shape[0]
    wcin = w * cin
    wch = w * ch
    h2, h4 = h // 2, h // 4
    w2c = (w // 2) * ch
    wqc = (w // 4) * ch
    M1, M2 = B * h, B * h2

    def sd(Y):
        z = jnp.zeros((Y.shape[0], 1, Y.shape[-1]), Y.dtype)
        return jnp.concatenate([z, Y[:, :-1, :]], axis=1)

    def su(Y):
        z = jnp.zeros((Y.shape[0], 1, Y.shape[-1]), Y.dtype)
        return jnp.concatenate([Y[:, 1:, :], z], axis=1)

    def half(X4):
        Bh = X4.shape[0]
        m1_, m2_ = Bh * h, Bh * h2
        Xd = jnp.stack([sd(X4[:, 3]), X4[:, 2], X4[:, 0], X4[:, 1]],
                       axis=1).reshape(m1_, wcin)
        Xu = jnp.stack([X4[:, 2], X4[:, 3], X4[:, 1], su(X4[:, 0])],
                       axis=1).reshape(m1_, wcin)
        X = X4.reshape(m1_, wcin)
        X3 = jnp.concatenate([Xd, X, Xu], axis=1).astype(bf16)

        acc1 = jnp.dot(X3, w1_ref[...], preferred_element_type=f32)
        acc1 = jnp.maximum(acc1 + b1_ref[...], 0.0).reshape(Bh, 2, h2, wch)
        rm = jnp.maximum(acc1[:, 0], acc1[:, 1])
        m1 = jnp.maximum(rm[..., :w2c], rm[..., w2c:]).astype(bf16)

        out = jnp.dot(m1.reshape(m2_, w2c), w2_ref[...],
                      preferred_element_type=f32)
        o0 = out[:, :w2c].reshape(Bh, 2, h4, w2c)
        o1 = out[:, w2c:2 * w2c].reshape(Bh, 2, h4, w2c)
        o2 = out[:, 2 * w2c:].reshape(Bh, 2, h4, w2c)
        dpart = jnp.stack([sd(o0[:, 1]), o0[:, 0]], axis=1)
        upart = jnp.stack([o2[:, 1], su(o2[:, 0])], axis=1)
        acc2 = jnp.maximum(o1 + dpart + upart + b2_ref[...], 0.0)
        rm2 = jnp.maximum(acc2[:, 0], acc2[:, 1])
        return jnp.maximum(rm2[..., :wqc], rm2[..., wqc:])

    X4 = x_ref[...].reshape(B, 4, h4, wcin)
    o_ref[...] = half(X4).astype(o_ref.dtype)


def _mlp_kernel(x_ref, w1_ref, b1_ref, w2_ref, b2_ref, o_ref):
    f32 = jnp.float32
    w1 = w1_ref[...].astype(jnp.bfloat16)
    hid = jnp.dot(x_ref[...], w1, preferred_element_type=f32)
    hid = jnp.maximum(hid + b1_ref[...], 0.0).astype(jnp.bfloat16)
    out = jnp.dot(hid, w2_ref[...], preferred_element_type=f32) + b2_ref[...]
    o_ref[...] = out


def _colperm(a, npix, ch):
    lead = a.shape[:-1]
    a = a.reshape(*lead, npix // 2, 2, ch)
    a = jnp.swapaxes(a, -3, -2)
    return a.reshape(*lead, npix * ch)


def _forward(x, bw1, b1row, bw2, b2row, w1p, b1p, w2p, b2p, *, num_classes):
    n, cin, h, w = x.shape
    wch = b1row.shape[1]
    ch = wch // w
    w2c = b2row.shape[1]
    wp = w // 2
    h4 = h // 4
    wqc = (w // 4) * ch
    hp = w1p.shape[1]
    cp = w2p.shape[1]
    bf16 = jnp.bfloat16

    xt = x.reshape(n, cin, h4, 2, 2, w).transpose(0, 4, 3, 2, 1, 5)
    xt = xt.reshape(n, h, cin * w).astype(bf16)
    w1cat = jnp.transpose(bw1.astype(bf16), (1, 0, 2, 3))
    w1cat = _colperm(w1cat.reshape(3 * cin * w, wch), w, ch)
    b1c = _colperm(b1row, w, ch)
    w2c3 = _colperm(bw2.astype(bf16), wp, ch)
    w2big = w2c3.transpose(1, 0, 2).reshape(w2c, 3 * w2c)
    b2c = _colperm(b2row, wp, ch)

    B = next(b for b in (32, 16, 8, 4, 2, 1) if n % b == 0)
    feats = pl.pallas_call(
        functools.partial(_conv_stack_kernel, h=h, w=w, cin=cin, ch=ch),
        out_shape=jax.ShapeDtypeStruct((n, h4, wqc), bf16),
        grid=(n // B,),
        in_specs=[
            pl.BlockSpec((B, h, cin * w), lambda i: (i, 0, 0)),
            pl.BlockSpec((3 * cin * w, wch), lambda i: (0, 0)),
            pl.BlockSpec((1, wch), lambda i: (0, 0)),
            pl.BlockSpec((w2c, 3 * w2c), lambda i: (0, 0)),
            pl.BlockSpec((1, w2c), lambda i: (0, 0)),
        ],
        out_specs=pl.BlockSpec((B, h4, wqc), lambda i: (i, 0, 0)),
        compiler_params=pltpu.CompilerParams(
            dimension_semantics=("parallel",)),
    )(xt, w1cat, b1c, w2big, b2c)

    flat = feats.reshape(n, h4 * wqc)
    mt = n
    logits = pl.pallas_call(
        _mlp_kernel,
        out_shape=jax.ShapeDtypeStruct((n, cp), jnp.float32),
        grid=(n // mt,),
        in_specs=[
            pl.BlockSpec((mt, h4 * wqc), lambda i: (i, 0)),
            pl.BlockSpec((h4 * wqc, hp), lambda i: (0, 0)),
            pl.BlockSpec((1, hp), lambda i: (0, 0)),
            pl.BlockSpec((hp, cp), lambda i: (0, 0)),
            pl.BlockSpec((1, cp), lambda i: (0, 0)),
        ],
        out_specs=pl.BlockSpec((mt, cp), lambda i: (i, 0)),
        compiler_params=pltpu.CompilerParams(
            dimension_semantics=("parallel",)),
    )(flat, w1p, b1p, w2p.astype(bf16), b2p)
    return {"out": logits[:, :num_classes]}


def kernel(x, bw1, b1row, bw2, b2row, w1p, b1p, w2p, b2p):
    return _forward(x, bw1, b1row, bw2, b2row, w1p, b1p, w2p, b2p,
                    num_classes=100)


# speedup vs baseline: 3.4118x; 1.0080x over previous
import functools

import jax
import jax.numpy as jnp
from jax.experimental import pallas as pl
from jax.experimental.pallas import tpu as pltpu


def _conv_stack_kernel(x_ref, w1_ref, b1_ref, w2_ref, b2_ref, o_ref, *,
                       h, w, cin, ch):
    f32 = jnp.float32
    bf16 = jnp.bfloat16
    B = x_ref.shape[0]
    wcin = w * cin
    wch = w * ch
    h2, h4 = h // 2, h // 4
    w2c = (w // 2) * ch
    wqc = (w // 4) * ch
    M1, M2 = B * h, B * h2

    def sd(Y):
        z = jnp.zeros((Y.shape[0], 1, Y.shape[-1]), Y.dtype)
        return jnp.concatenate([z, Y[:, :-1, :]], axis=1)

    def su(Y):
        z = jnp.zeros((Y.shape[0], 1, Y.shape[-1]), Y.dtype)
        return jnp.concatenate([Y[:, 1:, :], z], axis=1)

    def half(X4):
        Bh = X4.shape[0]
        m1_, m2_ = Bh * h, Bh * h2
        Xd = jnp.stack([sd(X4[:, 3]), X4[:, 2], X4[:, 0], X4[:, 1]],
                       axis=1).reshape(m1_, wcin)
        Xu = jnp.stack([X4[:, 2], X4[:, 3], X4[:, 1], su(X4[:, 0])],
                       axis=1).reshape(m1_, wcin)
        X = X4.reshape(m1_, wcin)
        X3 = jnp.concatenate([Xd, X, Xu], axis=1).astype(bf16)

        acc1 = jnp.dot(X3, w1_ref[...], preferred_element_type=f32)
        acc1 = jnp.maximum(acc1 + b1_ref[...], 0.0).reshape(Bh, 2, h2, wch)
        rm = jnp.maximum(acc1[:, 0], acc1[:, 1])
        m1 = jnp.maximum(rm[..., :w2c], rm[..., w2c:]).astype(bf16)

        m1f = m1.reshape(m2_, w2c)
        o0 = jnp.dot(m1f, w2_ref[0], preferred_element_type=f32)
        o1 = jnp.dot(m1f, w2_ref[1], preferred_element_type=f32)
        o2 = jnp.dot(m1f, w2_ref[2], preferred_element_type=f32)
        o0 = o0.reshape(Bh, 2, h4, w2c)
        o1 = o1.reshape(Bh, 2, h4, w2c)
        o2 = o2.reshape(Bh, 2, h4, w2c)
        dpart = jnp.stack([sd(o0[:, 1]), o0[:, 0]], axis=1)
        upart = jnp.stack([o2[:, 1], su(o2[:, 0])], axis=1)
        acc2 = jnp.maximum(o1 + dpart + upart + b2_ref[...], 0.0)
        rm2 = jnp.maximum(acc2[:, 0], acc2[:, 1])
        return jnp.maximum(rm2[..., :wqc], rm2[..., wqc:])

    X4 = x_ref[...].reshape(B, 4, h4, wcin)
    o_ref[...] = half(X4).astype(o_ref.dtype)


def _mlp_kernel(x_ref, w1_ref, b1_ref, w2_ref, b2_ref, o_ref):
    f32 = jnp.float32
    h4 = w1_ref.shape[0]
    acc = None
    for i2 in range(h4):
        d = jnp.dot(x_ref[:, i2, :], w1_ref[i2].astype(jnp.bfloat16),
                    preferred_element_type=f32)
        acc = d if acc is None else acc + d
    hid = jnp.maximum(acc + b1_ref[...], 0.0).astype(jnp.bfloat16)
    out = jnp.dot(hid, w2_ref[...], preferred_element_type=f32) + b2_ref[...]
    o_ref[...] = out


def _colperm(a, npix, ch):
    lead = a.shape[:-1]
    a = a.reshape(*lead, npix // 2, 2, ch)
    a = jnp.swapaxes(a, -3, -2)
    return a.reshape(*lead, npix * ch)


def _forward(x, bw1, b1row, bw2, b2row, w1p, b1p, w2p, b2p, *, num_classes):
    n, cin, h, w = x.shape
    wch = b1row.shape[1]
    ch = wch // w
    w2c = b2row.shape[1]
    wp = w // 2
    h4 = h // 4
    wqc = (w // 4) * ch
    hp = w1p.shape[1]
    cp = w2p.shape[1]
    bf16 = jnp.bfloat16

    xt = x.reshape(n, cin, h4, 2, 2, w).transpose(0, 4, 3, 2, 1, 5)
    xt = xt.reshape(n, h, cin * w).astype(bf16)
    w1cat = jnp.transpose(bw1.astype(bf16), (1, 0, 2, 3))
    w1cat = _colperm(w1cat.reshape(3 * cin * w, wch), w, ch)
    b1c = _colperm(b1row, w, ch)
    w2c3 = _colperm(bw2.astype(bf16), wp, ch)
    b2c = _colperm(b2row, wp, ch)

    B = next(b for b in (32, 16, 8, 4, 2, 1) if n % b == 0)
    feats = pl.pallas_call(
        functools.partial(_conv_stack_kernel, h=h, w=w, cin=cin, ch=ch),
        out_shape=jax.ShapeDtypeStruct((n, h4, wqc), bf16),
        grid=(n // B,),
        in_specs=[
            pl.BlockSpec((B, h, cin * w), lambda i: (i, 0, 0)),
            pl.BlockSpec((3 * cin * w, wch), lambda i: (0, 0)),
            pl.BlockSpec((1, wch), lambda i: (0, 0)),
            pl.BlockSpec((3, w2c, w2c), lambda i: (0, 0, 0)),
            pl.BlockSpec((1, w2c), lambda i: (0, 0)),
        ],
        out_specs=pl.BlockSpec((B, h4, wqc), lambda i: (i, 0, 0)),
        compiler_params=pltpu.CompilerParams(
            dimension_semantics=("parallel",)),
    )(xt, w1cat, b1c, w2c3, b2c)

    w1r = w1p.reshape(h4, wqc, hp)
    mt = n
    logits = pl.pallas_call(
        _mlp_kernel,
        out_shape=jax.ShapeDtypeStruct((n, cp), jnp.float32),
        grid=(n // mt,),
        in_specs=[
            pl.BlockSpec((mt, h4, wqc), lambda i: (i, 0, 0)),
            pl.BlockSpec((h4, wqc, hp), lambda i: (0, 0, 0)),
            pl.BlockSpec((1, hp), lambda i: (0, 0)),
            pl.BlockSpec((hp, cp), lambda i: (0, 0)),
            pl.BlockSpec((1, cp), lambda i: (0, 0)),
        ],
        out_specs=pl.BlockSpec((mt, cp), lambda i: (i, 0)),
        compiler_params=pltpu.CompilerParams(
            dimension_semantics=("parallel",)),
    )(feats, w1r, b1p, w2p.astype(bf16), b2p)
    return {"out": logits[:, :num_classes]}


def kernel(x, bw1, b1row, bw2, b2row, w1p, b1p, w2p, b2p):
    return _forward(x, bw1, b1row, bw2, b2row, w1p, b1p, w2p, b2p,
                    num_classes=100)


# speedup vs baseline: 3.5108x; 1.0290x over previous
import functools

import jax
import jax.numpy as jnp
from jax.experimental import pallas as pl
from jax.experimental.pallas import tpu as pltpu


def _conv_stack_kernel(x_ref, w1_ref, b1_ref, w2_ref, b2_ref, o_ref, *,
                       h, w, cin, ch):
    f32 = jnp.float32
    bf16 = jnp.bfloat16
    B = x_ref.shape[0]
    wcin = w * cin
    wch = w * ch
    h2, h4 = h // 2, h // 4
    w2c = (w // 2) * ch
    wqc = (w // 4) * ch
    M1, M2 = B * h, B * h2

    def sd(Y):
        z = jnp.zeros((Y.shape[0], 1, Y.shape[-1]), Y.dtype)
        return jnp.concatenate([z, Y[:, :-1, :]], axis=1)

    def su(Y):
        z = jnp.zeros((Y.shape[0], 1, Y.shape[-1]), Y.dtype)
        return jnp.concatenate([Y[:, 1:, :], z], axis=1)

    def half(X4):
        Bh = X4.shape[0]
        m1_, m2_ = Bh * h, Bh * h2
        Xd = jnp.stack([sd(X4[:, 3]), X4[:, 2], X4[:, 0], X4[:, 1]],
                       axis=1).reshape(m1_, wcin)
        Xu = jnp.stack([X4[:, 2], X4[:, 3], X4[:, 1], su(X4[:, 0])],
                       axis=1).reshape(m1_, wcin)
        X = X4.reshape(m1_, wcin)
        X3 = jnp.concatenate([Xd, X, Xu], axis=1).astype(bf16)

        acc1 = jnp.dot(X3, w1_ref[...], preferred_element_type=f32)
        acc1 = jnp.maximum(acc1 + b1_ref[...], 0.0).reshape(Bh, 2, h2, wch)
        rm = jnp.maximum(acc1[:, 0], acc1[:, 1])
        m1 = jnp.maximum(rm[..., :w2c], rm[..., w2c:]).astype(bf16)

        m1f = m1.reshape(m2_, w2c)
        o0 = jnp.dot(m1f, w2_ref[0], preferred_element_type=f32)
        o1 = jnp.dot(m1f, w2_ref[1], preferred_element_type=f32)
        o2 = jnp.dot(m1f, w2_ref[2], preferred_element_type=f32)
        o0 = o0.reshape(Bh, 2, h4, w2c)
        o1 = o1.reshape(Bh, 2, h4, w2c)
        o2 = o2.reshape(Bh, 2, h4, w2c)
        dpart = jnp.stack([sd(o0[:, 1]), o0[:, 0]], axis=1)
        upart = jnp.stack([o2[:, 1], su(o2[:, 0])], axis=1)
        acc2 = jnp.maximum(o1 + dpart + upart + b2_ref[...], 0.0)
        rm2 = jnp.maximum(acc2[:, 0], acc2[:, 1])
        return jnp.maximum(rm2[..., :wqc], rm2[..., wqc:])

    X4 = x_ref[...].reshape(B, 4, h4, wcin)
    o_ref[...] = half(X4).astype(o_ref.dtype)


def _mlp_kernel(x_ref, w1_ref, b1_ref, w2_ref, b2_ref, o_ref):
    f32 = jnp.float32
    h4 = w1_ref.shape[0]
    acc = None
    for i2 in range(h4):
        d = jnp.dot(x_ref[:, i2, :], w1_ref[i2].astype(jnp.bfloat16),
                    preferred_element_type=f32)
        acc = d if acc is None else acc + d
    hid = jnp.maximum(acc + b1_ref[...], 0.0).astype(jnp.bfloat16)
    out = jnp.dot(hid, w2_ref[...], preferred_element_type=f32) + b2_ref[...]
    o_ref[...] = out


def _colperm(a, npix, ch):
    lead = a.shape[:-1]
    a = a.reshape(*lead, npix // 2, 2, ch)
    a = jnp.swapaxes(a, -3, -2)
    return a.reshape(*lead, npix * ch)


def _forward(x, bw1, b1row, bw2, b2row, w1p, b1p, w2p, b2p, *, num_classes):
    n, cin, h, w = x.shape
    wch = b1row.shape[1]
    ch = wch // w
    w2c = b2row.shape[1]
    wp = w // 2
    h4 = h // 4
    wqc = (w // 4) * ch
    hp = w1p.shape[1]
    cp = w2p.shape[1]
    bf16 = jnp.bfloat16

    xt = x.reshape(n, cin, h4, 2, 2, w).transpose(0, 4, 3, 2, 1, 5)
    xt = xt.reshape(n, h, cin * w).astype(bf16)
    w1cat = jnp.transpose(bw1.astype(bf16), (1, 0, 2, 3))
    w1cat = _colperm(w1cat.reshape(3 * cin * w, wch), w, ch)
    b1c = _colperm(b1row, w, ch)
    w2c3 = _colperm(bw2.astype(bf16), wp, ch)
    b2c = _colperm(b2row, wp, ch)

    B = next(b for b in (64, 32, 16, 8, 4, 2, 1) if n % b == 0)
    feats = pl.pallas_call(
        functools.partial(_conv_stack_kernel, h=h, w=w, cin=cin, ch=ch),
        out_shape=jax.ShapeDtypeStruct((n, h4, wqc), bf16),
        grid=(n // B,),
        in_specs=[
            pl.BlockSpec((B, h, cin * w), lambda i: (i, 0, 0)),
            pl.BlockSpec((3 * cin * w, wch), lambda i: (0, 0)),
            pl.BlockSpec((1, wch), lambda i: (0, 0)),
            pl.BlockSpec((3, w2c, w2c), lambda i: (0, 0, 0)),
            pl.BlockSpec((1, w2c), lambda i: (0, 0)),
        ],
        out_specs=pl.BlockSpec((B, h4, wqc), lambda i: (i, 0, 0)),
        compiler_params=pltpu.CompilerParams(
            dimension_semantics=("parallel",)),
    )(xt, w1cat, b1c, w2c3, b2c)

    w1r = w1p.reshape(h4, wqc, hp)
    mt = n // 2 if n % 2 == 0 else n
    logits = pl.pallas_call(
        _mlp_kernel,
        out_shape=jax.ShapeDtypeStruct((n, cp), jnp.float32),
        grid=(n // mt,),
        in_specs=[
            pl.BlockSpec((mt, h4, wqc), lambda i: (i, 0, 0)),
            pl.BlockSpec((h4, wqc, hp), lambda i: (0, 0, 0)),
            pl.BlockSpec((1, hp), lambda i: (0, 0)),
            pl.BlockSpec((hp, cp), lambda i: (0, 0)),
            pl.BlockSpec((1, cp), lambda i: (0, 0)),
        ],
        out_specs=pl.BlockSpec((mt, cp), lambda i: (i, 0)),
        compiler_params=pltpu.CompilerParams(
            dimension_semantics=("parallel",)),
    )(feats, w1r, b1p, w2p.astype(bf16), b2p)
    return {"out": logits[:, :num_classes]}


def kernel(x, bw1, b1row, bw2, b2row, w1p, b1p, w2p, b2p):
    return _forward(x, bw1, b1row, bw2, b2row, w1p, b1p, w2p, b2p,
                    num_classes=100)
